# Initial kernel scaffold; baseline (speedup 1.0000x reference)
import jax
import jax.numpy as jnp
from jax.experimental import pallas as pl


def kernel(node_inputs, src_ids, dst_ids, W1, b1, W2, b2, W3, b3, Wout, bout, W_ih, W_hh, b_ih, b_hh):
    raise NotImplementedError("write your pallas kernel here")


# speedup vs baseline: 3.3267x; 3.3267x over previous
import functools

import jax
import jax.numpy as jnp
from jax import lax
from jax.experimental import pallas as pl
from jax.experimental.pallas import tpu as pltpu
from jax.experimental.pallas import tpu_sc as plsc

N_NODES = 10000
N_EDGES = 160000
N_ITERS = 7
DH = 10
DE = 11
DIN = 9
MLP_H = 96

NP = 10016
SINK = N_NODES
NW = 32
EPW = 5120
E_PAD = NW * EPW
RPT = NP // 16

@functools.cache
def _mesh():
    return plsc.VectorSubcoreMesh(core_axis_name="c", subcore_axis_name="s")


def _gather_body(state_hbm, src_hbm, dst_hbm, xs_hbm, xd_hbm, idx_v, rows_v, sem):
    wid = lax.axis_index("s") * 2 + lax.axis_index("c")
    pltpu.sync_copy(src_hbm.at[wid], idx_v)
    pltpu.async_copy(state_hbm.at[idx_v], rows_v, sem).wait()
    pltpu.sync_copy(rows_v, xs_hbm.at[wid])
    pltpu.sync_copy(dst_hbm.at[wid], idx_v)
    pltpu.async_copy(state_hbm.at[idx_v], rows_v, sem).wait()
    pltpu.sync_copy(rows_v, xd_hbm.at[wid])


@jax.jit
def _sc_gather(state, src3, dst3):
    return pl.kernel(
        _gather_body,
        out_type=[
            jax.ShapeDtypeStruct((NW, EPW, 16), jnp.float32),
            jax.ShapeDtypeStruct((NW, EPW, 16), jnp.float32),
        ],
        mesh=_mesh(),
        compiler_params=pltpu.CompilerParams(use_tc_tiling_on_sc=False),
        scratch_types=[
            pltpu.VMEM((EPW,), jnp.int32),
            pltpu.VMEM((EPW, 16), jnp.float32),
            pltpu.SemaphoreType.DMA,
        ],
    )(state, src3, dst3)


def _scatter_body(msgs_hbm, dst_hbm, out_hbm, idx_v, rows_v, zb_v, sem, agg_sh):
    c = lax.axis_index("c")
    s = lax.axis_index("s")
    wid = s * 2 + c

    def _zero(i, carry):
        zb_v[i] = jnp.zeros((16,), jnp.float32)
        return carry

    lax.fori_loop(0, RPT, _zero, 0)
    pltpu.sync_copy(zb_v, agg_sh.at[pl.ds(s * RPT, RPT)])
    plsc.subcore_barrier()
    pltpu.sync_copy(msgs_hbm.at[wid], rows_v)
    pltpu.sync_copy(dst_hbm.at[wid], idx_v)
    pltpu.sync_copy(rows_v, agg_sh.at[idx_v], add=True)
    plsc.subcore_barrier()
    pltpu.sync_copy(agg_sh.at[pl.ds(s * RPT, RPT)],
                    out_hbm.at[c, pl.ds(s * RPT, RPT)])


@jax.jit
def _sc_scatter(msgs4, dst3):
    return pl.kernel(
        _scatter_body,
        out_type=jax.ShapeDtypeStruct((2, NP, 16), jnp.float32),
        mesh=_mesh(),
        compiler_params=pltpu.CompilerParams(use_tc_tiling_on_sc=False),
        scratch_types=[
            pltpu.VMEM((EPW,), jnp.int32),
            pltpu.VMEM((EPW, 16), jnp.float32),
            pltpu.VMEM((RPT, 16), jnp.float32),
            pltpu.SemaphoreType.DMA,
            pltpu.VMEM_SHARED((NP, 16), jnp.float32),
        ],
    )(msgs4, dst3)


_BE = 4096


def _mlp_body(xs_ref, xd_ref, w1a, w1b, b1, w2, b2, w3, b3, out_ref):
    h = jnp.dot(xs_ref[...], w1a[...], preferred_element_type=jnp.float32)
    h = h + jnp.dot(xd_ref[...], w1b[...], preferred_element_type=jnp.float32)
    h = jax.nn.relu(h + b1[...])
    h = jax.nn.relu(jnp.dot(h, w2[...], preferred_element_type=jnp.float32) + b2[...])
    out_ref[...] = jnp.dot(h, w3[...], preferred_element_type=jnp.float32) + b3[...]


@jax.jit
def _tc_mlp(xs, xd, w1a, w1b, b1, w2, b2, w3, b3):
    n_blk = E_PAD // _BE
    full = lambda i: (0, 0)
    return pl.pallas_call(
        _mlp_body,
        grid=(n_blk,),
        in_specs=[
            pl.BlockSpec((_BE, 16), lambda i: (i, 0)),
            pl.BlockSpec((_BE, 16), lambda i: (i, 0)),
            pl.BlockSpec((16, MLP_H), full),
            pl.BlockSpec((16, MLP_H), full),
            pl.BlockSpec((1, MLP_H), full),
            pl.BlockSpec((MLP_H, MLP_H), full),
            pl.BlockSpec((1, MLP_H), full),
            pl.BlockSpec((MLP_H, 16), full),
            pl.BlockSpec((1, 16), full),
        ],
        out_specs=pl.BlockSpec((_BE, 16), lambda i: (i, 0)),
        out_shape=jax.ShapeDtypeStruct((E_PAD, 16), jnp.float32),
    )(xs, xd, w1a, w1b, b1, w2, b2, w3, b3)


def _update_body(agg2_ref, st_ref, ni_ref, wia, wib, whh, bih, bhh, wout, bout,
                 ns_ref, out_ref):
    agg = agg2_ref[0] + agg2_ref[1]
    st = st_ref[...]
    gx = (jnp.dot(agg, wia[...], preferred_element_type=jnp.float32)
          + jnp.dot(ni_ref[...], wib[...], preferred_element_type=jnp.float32)
          + bih[...])
    gh = jnp.dot(st, whh[...], preferred_element_type=jnp.float32) + bhh[...]
    r = jax.nn.sigmoid(gx[:, 0:16] + gh[:, 0:16])
    z = jax.nn.sigmoid(gx[:, 16:32] + gh[:, 16:32])
    n = jnp.tanh(gx[:, 32:48] + r * gh[:, 32:48])
    ns = (1.0 - z) * n + z * st
    ns_ref[...] = ns
    logits = jnp.dot(ns, wout[...], preferred_element_type=jnp.float32) + bout[...]
    col = lax.broadcasted_iota(jnp.int32, logits.shape, 1)
    logits = jnp.where(col < 9, logits, -1e30)
    m = jnp.max(logits, axis=1, keepdims=True)
    e = jnp.exp(logits - m)
    out_ref[...] = e / jnp.sum(e, axis=1, keepdims=True)


@jax.jit
def _tc_update(agg2, state, ni, wia, wib, whh, bih, bhh, wout, bout):
    full = lambda: (0, 0)
    return pl.pallas_call(
        _update_body,
        out_shape=[
            jax.ShapeDtypeStruct((NP, 16), jnp.float32),
            jax.ShapeDtypeStruct((NP, 16), jnp.float32),
        ],
    )(agg2, state, ni, wia, wib, whh, bih, bhh, wout, bout)


def _prep_idx(ids):
    ids = ids.reshape(NW, N_EDGES // NW)
    pad = jnp.full((NW, EPW - N_EDGES // NW), SINK, dtype=jnp.int32)
    return jnp.concatenate([ids, pad], axis=1).reshape(NW, EPW)


def _pad2(a, rows, cols):
    return jnp.zeros((rows, cols), a.dtype).at[: a.shape[0], : a.shape[1]].set(a)


def _gate_pad(wt, in_real):
    out = jnp.zeros((16, 48), wt.dtype)
    for g in range(3):
        out = out.at[:in_real, g * 16:g * 16 + DH].set(wt[:, g * DH:(g + 1) * DH])
    return out


def kernel(node_inputs, src_ids, dst_ids, W1, b1, W2, b2, W3, b3, Wout, bout,
           W_ih, W_hh, b_ih, b_hh):
    f32 = jnp.float32
    src3 = _prep_idx(src_ids)
    dst3 = _prep_idx(dst_ids)
    ni = _pad2(node_inputs.astype(f32), NP, 16)

    w1t = W1.T
    w1a = _pad2(w1t[:DH], 16, MLP_H)
    w1b = _pad2(w1t[DH:], 16, MLP_H)
    b1r = b1.reshape(1, MLP_H)
    w2t = W2.T
    b2r = b2.reshape(1, MLP_H)
    w3t = _pad2(W3.T, MLP_H, 16)
    b3r = _pad2(b3.reshape(1, DE), 1, 16)

    wiht = W_ih.T
    wia = _gate_pad(wiht[:DE], DE)
    wib = _gate_pad(wiht[DE:], DIN)
    whh = _gate_pad(W_hh.T, DH)
    bih = jnp.zeros((1, 48), f32)
    bhh = jnp.zeros((1, 48), f32)
    for g in range(3):
        bih = bih.at[0, g * 16:g * 16 + DH].set(b_ih[g * DH:(g + 1) * DH])
        bhh = bhh.at[0, g * 16:g * 16 + DH].set(b_hh[g * DH:(g + 1) * DH])
    woutt = _pad2(Wout.T, 16, 16)
    boutr = _pad2(bout.reshape(1, DOUT := 9), 1, 16)

    state = jnp.zeros((NP, 16), f32)
    outs = []
    for _ in range(N_ITERS):
        xs4, xd4 = _sc_gather(state, src3, dst3)
        msgs = _tc_mlp(xs4.reshape(E_PAD, 16), xd4.reshape(E_PAD, 16),
                       w1a, w1b, b1r, w2t, b2r, w3t, b3r)
        agg2 = _sc_scatter(msgs.reshape(NW, EPW, 16), dst3)
        state, out_i = _tc_update(agg2, state, ni, wia, wib, whh, bih, bhh,
                                  woutt, boutr)
        outs.append(out_i)
    total = jnp.stack(outs, axis=0)
    return total[:, :N_NODES, :9]
